# baseline (speedup 1.0000x reference)
import functools

import jax
import jax.numpy as jnp
from jax import lax
from jax.experimental import pallas as pl
from jax.experimental.pallas import tpu as pltpu
from jax.experimental.pallas import tpu_sc as plsc

NUM_CONT = 13
NUM_CAT = 26
VOCAB = 100000
EMB_DIM = 16
BATCH = 16384
EMB_W = NUM_CAT * EMB_DIM
OUT_W = NUM_CONT + EMB_W

PACK = 128 // EMB_DIM
NSLAB = NUM_CAT * VOCAB // PACK

NC = 2
NS = 16
NW = NC * NS
ROWS_PER_W = BATCH // NW
C = 128
NCHUNK = ROWS_PER_W // C
NCHUNKS_TOTAL = BATCH // C
NG = C // 16

TC_BLOCK = 1024


def _gather_body(sid_hbm, off_hbm, tab_hbm, emb_hbm,
                 sid_v, off_v, slab_v, row_v, gsem, osem):
    core = lax.axis_index("c")
    sub = lax.axis_index("s")
    wid = sub * NC + core
    iota16 = lax.iota(jnp.int32, 16)

    def chunk_body(cblk, carry):
        chunk = wid * NCHUNK + cblk
        r0 = chunk * C

        pltpu.sync_copy(sid_hbm.at[chunk], sid_v)
        pltpu.sync_copy(off_hbm.at[chunk], off_v)

        def fire(j):
            p = j % 3
            for k in range(NG):
                iv = sid_v[j, pl.ds(16 * k, 16)]
                pltpu.async_copy(tab_hbm.at[iv],
                                 slab_v.at[p, pl.ds(16 * k, 16), :],
                                 gsem.at[p])

        fire(0)
        fire(1)

        def col_body(j, c):
            @pl.when(j + 2 < NUM_CAT)
            def _():
                fire(j + 2)

            p = j % 3
            for k in range(NG):
                iv = sid_v[j, pl.ds(16 * k, 16)]
                pltpu.make_async_copy(tab_hbm.at[iv],
                                      slab_v.at[p, pl.ds(16 * k, 16), :],
                                      gsem.at[p]).wait()

            slab = slab_v.at[p]
            colbase = EMB_DIM * j
            for k in range(NG):
                rvec = k * 16 + iota16
                offv = off_v[j, pl.ds(16 * k, 16)]
                for cc in range(EMB_DIM):
                    val = plsc.load_gather(slab, [rvec, offv + cc])
                    cvec = jnp.zeros((16,), jnp.int32) + (colbase + cc)
                    plsc.store_scatter(row_v, [rvec, cvec], val)
            return c
        lax.fori_loop(0, NUM_CAT, col_body, 0)

        pltpu.async_copy(row_v, emb_hbm.at[pl.ds(r0, C), :], osem)
        pltpu.make_async_copy(row_v, emb_hbm.at[pl.ds(r0, C), :],
                              osem).wait()
        return carry

    lax.fori_loop(0, NCHUNK, chunk_body, 0)


_gather_call = functools.partial(
    pl.kernel,
    mesh=plsc.VectorSubcoreMesh(core_axis_name="c", subcore_axis_name="s"),
    out_type=jax.ShapeDtypeStruct((BATCH, EMB_W), jnp.float32),
    compiler_params=pltpu.CompilerParams(needs_layout_passes=False),
    scratch_types=[
        pltpu.VMEM((NUM_CAT, C), jnp.int32),
        pltpu.VMEM((NUM_CAT, C), jnp.int32),
        pltpu.VMEM((3, C, 128), jnp.float32),
        pltpu.VMEM((C, EMB_W), jnp.float32),
        pltpu.SemaphoreType.DMA((3,)),
        pltpu.SemaphoreType.DMA,
    ],
)(_gather_body)


def _interleave_body(x_ref, emb_ref, out_ref):
    cont = x_ref[:, :NUM_CONT].astype(jnp.float32)
    out_ref[...] = jnp.concatenate([cont, emb_ref[...]], axis=1)


def _interleave(x, emb):
    return pl.pallas_call(
        _interleave_body,
        grid=(BATCH // TC_BLOCK,),
        in_specs=[
            pl.BlockSpec((TC_BLOCK, NUM_CONT + NUM_CAT), lambda i: (i, 0)),
            pl.BlockSpec((TC_BLOCK, EMB_W), lambda i: (i, 0)),
        ],
        out_specs=pl.BlockSpec((TC_BLOCK, OUT_W), lambda i: (i, 0)),
        out_shape=jax.ShapeDtypeStruct((BATCH, OUT_W), jnp.float32),
    )(x, emb)


def kernel(x, tables):
    x = x.astype(jnp.int32)
    tab = tables.reshape(NSLAB, 128)
    flat = x[:, NUM_CONT:] + jnp.arange(
        NUM_CAT, dtype=jnp.int32)[None, :] * VOCAB
    sid = jnp.right_shift(flat, 3)
    off = jnp.left_shift(jnp.bitwise_and(flat, 7), 4)
    sid = sid.reshape(NCHUNKS_TOTAL, C, NUM_CAT).transpose(0, 2, 1)
    off = off.reshape(NCHUNKS_TOTAL, C, NUM_CAT).transpose(0, 2, 1)
    emb = _gather_call(sid, off, tab)
    return _interleave(x, emb)


# speedup vs baseline: 1.0602x; 1.0602x over previous
import functools

import jax
import jax.numpy as jnp
from jax import lax
from jax.experimental import pallas as pl
from jax.experimental.pallas import tpu as pltpu
from jax.experimental.pallas import tpu_sc as plsc

NUM_CONT = 13
NUM_CAT = 26
VOCAB = 100000
EMB_DIM = 16
BATCH = 16384
EMB_W = NUM_CAT * EMB_DIM
OUT_W = NUM_CONT + EMB_W

PACK = 128 // EMB_DIM
NSLAB = NUM_CAT * VOCAB // PACK

NC = 2
NS = 16
NW = NC * NS
ROWS_PER_W = BATCH // NW
C = 128
NCHUNK = ROWS_PER_W // C
NCHUNKS_TOTAL = BATCH // C
NG = C // 16

TC_BLOCK = 1024


def _gather_body(sid_hbm, off_hbm, tab_hbm, emb_hbm,
                 sid_v, off_v, slab_v, row_v, gsem, osem):
    core = lax.axis_index("c")
    sub = lax.axis_index("s")
    wid = sub * NC + core
    iota16 = lax.iota(jnp.int32, 16)

    def chunk_body(cblk, carry):
        chunk = wid * NCHUNK + cblk
        r0 = chunk * C

        pltpu.sync_copy(sid_hbm.at[chunk], sid_v)
        pltpu.sync_copy(off_hbm.at[chunk], off_v)

        def fire(g):
            p = g % 3
            for k in range(NG):
                iv = sid_v[g, pl.ds(16 * k, 16)]
                pltpu.async_copy(tab_hbm.at[iv],
                                 slab_v.at[p, pl.ds(16 * k, 16), :],
                                 gsem.at[p])

        fire(0)
        fire(1)

        def grp_body(g, c):
            @pl.when(g + 2 < NUM_CAT)
            def _():
                fire(g + 2)

            p = g % 3
            for k in range(NG):
                iv = sid_v[g, pl.ds(16 * k, 16)]
                pltpu.make_async_copy(tab_hbm.at[iv],
                                      slab_v.at[p, pl.ds(16 * k, 16), :],
                                      gsem.at[p]).wait()

            slab = slab_v.at[p]
            for k in range(NG):
                rvec = k * 16 + iota16
                flatv = g * 128 + k * 16 + iota16
                brow = lax.div(flatv, NUM_CAT)
                cbase = lax.rem(flatv, NUM_CAT) * EMB_DIM
                offv = off_v[g, pl.ds(16 * k, 16)]
                for cc in range(EMB_DIM):
                    val = plsc.load_gather(slab, [rvec, offv + cc])
                    plsc.store_scatter(row_v, [brow, cbase + cc], val)
            return c
        lax.fori_loop(0, NUM_CAT, grp_body, 0)

        pltpu.async_copy(row_v, emb_hbm.at[pl.ds(r0, C), :], osem)
        pltpu.make_async_copy(row_v, emb_hbm.at[pl.ds(r0, C), :],
                              osem).wait()
        return carry

    lax.fori_loop(0, NCHUNK, chunk_body, 0)


_gather_call = functools.partial(
    pl.kernel,
    mesh=plsc.VectorSubcoreMesh(core_axis_name="c", subcore_axis_name="s"),
    out_type=jax.ShapeDtypeStruct((BATCH, EMB_W), jnp.float32),
    compiler_params=pltpu.CompilerParams(needs_layout_passes=False),
    scratch_types=[
        pltpu.VMEM((NUM_CAT, C), jnp.int32),
        pltpu.VMEM((NUM_CAT, C), jnp.int32),
        pltpu.VMEM((3, C, 128), jnp.float32),
        pltpu.VMEM((C, EMB_W), jnp.float32),
        pltpu.SemaphoreType.DMA((3,)),
        pltpu.SemaphoreType.DMA,
    ],
)(_gather_body)


def _interleave_body(x_ref, emb_ref, out_ref):
    cont = x_ref[:, :NUM_CONT].astype(jnp.float32)
    out_ref[...] = jnp.concatenate([cont, emb_ref[...]], axis=1)


def _interleave(x, emb):
    return pl.pallas_call(
        _interleave_body,
        grid=(BATCH // TC_BLOCK,),
        in_specs=[
            pl.BlockSpec((TC_BLOCK, NUM_CONT + NUM_CAT), lambda i: (i, 0)),
            pl.BlockSpec((TC_BLOCK, EMB_W), lambda i: (i, 0)),
        ],
        out_specs=pl.BlockSpec((TC_BLOCK, OUT_W), lambda i: (i, 0)),
        out_shape=jax.ShapeDtypeStruct((BATCH, OUT_W), jnp.float32),
    )(x, emb)


def kernel(x, tables):
    x = x.astype(jnp.int32)
    tab = tables.reshape(NSLAB, 128)
    flat = x[:, NUM_CONT:] + jnp.arange(
        NUM_CAT, dtype=jnp.int32)[None, :] * VOCAB
    sid = jnp.right_shift(flat, 3)
    off = jnp.left_shift(jnp.bitwise_and(flat, 7), 4)
    sid = sid.reshape(NCHUNKS_TOTAL, NUM_CAT, 128)
    off = off.reshape(NCHUNKS_TOTAL, NUM_CAT, 128)
    emb = _gather_call(sid, off, tab)
    return _interleave(x, emb)


# speedup vs baseline: 1.0668x; 1.0063x over previous
import functools

import jax
import jax.numpy as jnp
from jax import lax
from jax.experimental import pallas as pl
from jax.experimental.pallas import tpu as pltpu
from jax.experimental.pallas import tpu_sc as plsc

NUM_CONT = 13
NUM_CAT = 26
VOCAB = 100000
EMB_DIM = 16
BATCH = 16384
EMB_W = NUM_CAT * EMB_DIM
OUT_W = NUM_CONT + EMB_W

PACK = 128 // EMB_DIM
NSLAB = NUM_CAT * VOCAB // PACK

NC = 2
NS = 16
NW = NC * NS
ROWS_PER_W = BATCH // NW
C = 128
NCHUNK = ROWS_PER_W // C
NCHUNKS_TOTAL = BATCH // C
NG = C // 16

TC_BLOCK = 1024


def _gather_body(sid_hbm, off_hbm, tab_hbm, emb_hbm,
                 sid_v, off_v, slab_v, row_v, gsem, osem):
    core = lax.axis_index("c")
    sub = lax.axis_index("s")
    wid = sub * NC + core
    iota16 = lax.iota(jnp.int32, 16)

    def chunk_body(cblk, carry):
        chunk = wid * NCHUNK + cblk
        r0 = chunk * C

        pltpu.sync_copy(sid_hbm.at[chunk], sid_v)
        pltpu.sync_copy(off_hbm.at[chunk], off_v)

        def fire(g):
            p = g % 3
            for k in range(NG):
                iv = sid_v[g, pl.ds(16 * k, 16)]
                pltpu.async_copy(tab_hbm.at[iv],
                                 slab_v.at[p, pl.ds(16 * k, 16), :],
                                 gsem.at[p])

        fire(0)
        fire(1)

        def grp_body(g, c):
            @pl.when(g + 2 < NUM_CAT)
            def _():
                fire(g + 2)

            p = g % 3
            for k in range(NG):
                iv = sid_v[g, pl.ds(16 * k, 16)]
                pltpu.make_async_copy(tab_hbm.at[iv],
                                      slab_v.at[p, pl.ds(16 * k, 16), :],
                                      gsem.at[p]).wait()

            slab = slab_v.at[p]
            for k in range(NG):
                rvec = k * 16 + iota16
                flatv = g * 128 + k * 16 + iota16
                brow = lax.div(flatv, NUM_CAT)
                cbase = lax.rem(flatv, NUM_CAT) * EMB_DIM
                offv = off_v[g, pl.ds(16 * k, 16)]
                for cc in range(EMB_DIM):
                    val = plsc.load_gather(slab, [rvec, offv + cc])
                    plsc.store_scatter(row_v, [brow, cbase + cc], val)
            return c
        lax.fori_loop(0, NUM_CAT, grp_body, 0)

        pltpu.async_copy(row_v, emb_hbm.at[pl.ds(r0, C), :], osem)
        pltpu.make_async_copy(row_v, emb_hbm.at[pl.ds(r0, C), :],
                              osem).wait()
        return carry

    lax.fori_loop(0, NCHUNK, chunk_body, 0)


_gather_call = functools.partial(
    pl.kernel,
    mesh=plsc.VectorSubcoreMesh(core_axis_name="c", subcore_axis_name="s"),
    out_type=jax.ShapeDtypeStruct((BATCH, EMB_W), jnp.float32),
    compiler_params=pltpu.CompilerParams(needs_layout_passes=False),
    scratch_types=[
        pltpu.VMEM((NUM_CAT, C), jnp.int32),
        pltpu.VMEM((NUM_CAT, C), jnp.int32),
        pltpu.VMEM((3, C, 128), jnp.float32),
        pltpu.VMEM((C, EMB_W), jnp.float32),
        pltpu.SemaphoreType.DMA((3,)),
        pltpu.SemaphoreType.DMA,
    ],
)(_gather_body)


def _interleave_body(x_ref, emb_ref, out_ref):
    cont = x_ref[:, :NUM_CONT].astype(jnp.float32)
    out_ref[...] = jnp.concatenate([cont, emb_ref[...]], axis=1)


def _interleave(x, emb):
    return pl.pallas_call(
        _interleave_body,
        grid=(BATCH // TC_BLOCK,),
        in_specs=[
            pl.BlockSpec((TC_BLOCK, NUM_CONT + NUM_CAT), lambda i: (i, 0)),
            pl.BlockSpec((TC_BLOCK, EMB_W), lambda i: (i, 0)),
        ],
        out_specs=pl.BlockSpec((TC_BLOCK, OUT_W), lambda i: (i, 0)),
        out_shape=jax.ShapeDtypeStruct((BATCH, OUT_W), jnp.float32),
    )(x, emb)


def kernel(x, tables):
    x = x.astype(jnp.int32)
    tab = tables.reshape(NSLAB, 128)
    flat = x[:, NUM_CONT:] + jnp.arange(
        NUM_CAT, dtype=jnp.int32)[None, :] * VOCAB
    sid = jnp.right_shift(flat, 3)
    off = jnp.left_shift(jnp.bitwise_and(flat, 7), 4)
    sid = sid.reshape(NCHUNKS_TOTAL, NUM_CAT, 128)
    off = off.reshape(NCHUNKS_TOTAL, NUM_CAT, 128)
    emb = _gather_call(sid, off, tab)
    cont = x[:, :NUM_CONT].astype(jnp.float32)
    return jnp.concatenate([cont, emb], axis=1)


# speedup vs baseline: 1.0678x; 1.0009x over previous
import functools

import jax
import jax.numpy as jnp
from jax import lax
from jax.experimental import pallas as pl
from jax.experimental.pallas import tpu as pltpu
from jax.experimental.pallas import tpu_sc as plsc

NUM_CONT = 13
NUM_CAT = 26
VOCAB = 100000
EMB_DIM = 16
BATCH = 16384
EMB_W = NUM_CAT * EMB_DIM
OUT_W = NUM_CONT + EMB_W

PACK = 128 // EMB_DIM
NSLAB = NUM_CAT * VOCAB // PACK

NC = 2
NS = 16
NW = NC * NS
ROWS_PER_W = BATCH // NW
C = 128
NCHUNK = ROWS_PER_W // C
NCHUNKS_TOTAL = BATCH // C
NG = C // 16

TC_BLOCK = 1024


def _gather_body(sid_hbm, off_hbm, tab_hbm, emb_hbm,
                 sid_v, off_v, slab_v, row_v, gsem, osem):
    core = lax.axis_index("c")
    sub = lax.axis_index("s")
    wid = sub * NC + core
    iota16 = lax.iota(jnp.int32, 16)

    def chunk_body(cblk, carry):
        chunk = wid * NCHUNK + cblk
        r0 = chunk * C

        pltpu.sync_copy(sid_hbm.at[chunk], sid_v)
        pltpu.sync_copy(off_hbm.at[chunk], off_v)

        def fire(g):
            p = g % 3
            for k in range(NG):
                iv = sid_v[g, pl.ds(16 * k, 16)]
                pltpu.async_copy(tab_hbm.at[iv],
                                 slab_v.at[p, pl.ds(16 * k, 16), :],
                                 gsem.at[p])

        fire(0)
        fire(1)

        def grp_body(g, c):
            @pl.when(g + 2 < NUM_CAT)
            def _():
                fire(g + 2)

            p = g % 3
            for k in range(NG):
                iv = sid_v[g, pl.ds(16 * k, 16)]
                pltpu.make_async_copy(tab_hbm.at[iv],
                                      slab_v.at[p, pl.ds(16 * k, 16), :],
                                      gsem.at[p]).wait()

            slab = slab_v.at[p]
            for k in range(NG):
                rvec = k * 16 + iota16
                flatv = g * 128 + k * 16 + iota16
                brow = lax.div(flatv, NUM_CAT)
                cbase = lax.rem(flatv, NUM_CAT) * EMB_DIM
                offv = off_v[g, pl.ds(16 * k, 16)]
                for cc in range(EMB_DIM):
                    val = plsc.load_gather(slab, [rvec, offv + cc])
                    plsc.store_scatter(row_v, [brow, cbase + cc], val)
            return c
        lax.fori_loop(0, NUM_CAT, grp_body, 0)

        pltpu.async_copy(row_v, emb_hbm.at[pl.ds(r0, C), :], osem)
        pltpu.make_async_copy(row_v, emb_hbm.at[pl.ds(r0, C), :],
                              osem).wait()
        return carry

    lax.fori_loop(0, NCHUNK, chunk_body, 0)


_gather_call = functools.partial(
    pl.kernel,
    mesh=plsc.VectorSubcoreMesh(core_axis_name="c", subcore_axis_name="s"),
    out_type=jax.ShapeDtypeStruct((BATCH, EMB_W), jnp.float32),
    compiler_params=pltpu.CompilerParams(needs_layout_passes=False),
    scratch_types=[
        pltpu.VMEM((NUM_CAT, C), jnp.int32),
        pltpu.VMEM((NUM_CAT, C), jnp.int32),
        pltpu.VMEM((3, C, 128), jnp.float32),
        pltpu.VMEM((C, EMB_W), jnp.float32),
        pltpu.SemaphoreType.DMA((3,)),
        pltpu.SemaphoreType.DMA,
    ],
)(_gather_body)


def kernel(x, tables):
    x = x.astype(jnp.int32)
    tab = tables.reshape(NSLAB, 128)
    flat = x[:, NUM_CONT:] + jnp.arange(
        NUM_CAT, dtype=jnp.int32)[None, :] * VOCAB
    sid = jnp.right_shift(flat, 3)
    off = jnp.left_shift(jnp.bitwise_and(flat, 7), 4)
    sid = sid.reshape(NCHUNKS_TOTAL, NUM_CAT, 128)
    off = off.reshape(NCHUNKS_TOTAL, NUM_CAT, 128)
    emb = _gather_call(sid, off, tab)
    cont = x[:, :NUM_CONT].astype(jnp.float32)
    return jnp.concatenate([cont, emb], axis=1)
